# Initial kernel scaffold; baseline (speedup 1.0000x reference)
import jax
import jax.numpy as jnp
from jax.experimental import pallas as pl


def kernel(meta_k, node_emb, W_f, W_hyper, W_att, att_i_o, att_j_o, edge_index):
    raise NotImplementedError("write your pallas kernel here")


# speedup vs baseline: 14.6316x; 14.6316x over previous
import functools

import jax
import jax.numpy as jnp
from jax import lax
from jax.experimental import pallas as pl
from jax.experimental.pallas import tpu as pltpu
from jax.experimental.pallas import tpu_sc as plsc

N_NODES = 10000
N_EDGES = 320000
D_IN = 128
D_OUT = 64
D_HYPER = 128
D_META = 64
SLOPE = 0.2

BLK = 400
OC = 8

NC = 2
NS = 16
NW = NC * NS
SUB = 80
ROWS_TOT = 4096
E_PAD = ROWS_TOT * SUB
ROWS_W = ROWS_TOT // NW
KSUB = 8
CHUNK = SUB * KSUB
NCHUNK = ROWS_W // KSUB
VEC = 16


def _hyper_body(x_ref, m_ref, wf_ref, g2_ref, wa_ref, aio_ref, ajo_ref,
                xt_ref, abc_ref):
    x = x_ref[...]
    xm = jnp.concatenate([x, m_ref[...]], axis=1)
    f = jnp.tanh(jnp.dot(xm, wf_ref[...], preferred_element_type=jnp.float32))
    parts = []
    ow = D_OUT // OC
    for oc in range(OC):
        g = jnp.dot(x, g2_ref[:, oc * ow * D_HYPER:(oc + 1) * ow * D_HYPER],
                    preferred_element_type=jnp.float32)
        g3 = g.reshape(BLK, ow, D_HYPER)
        parts.append((g3 * f[:, None, :]).sum(-1))
    xt = jnp.concatenate(parts, axis=1)
    ain = jnp.dot(f, wa_ref[...], preferred_element_type=jnp.float32)
    ai = (xt * aio_ref[...] * ain).sum(-1)
    aj = (xt * ajo_ref[...] * ain).sum(-1)
    s = ai + aj
    c = jnp.where(s > 0, s, s * SLOPE)
    z = jnp.zeros_like(ai)
    xt_ref[...] = xt
    abc_ref[...] = jnp.stack([ai, aj, c, z, z, z, z, z], axis=1)


def _run_hyper(x, meta_k, wfT, g2, waT, aio, ajo):
    grid = N_NODES // BLK
    return pl.pallas_call(
        _hyper_body,
        grid=(grid,),
        in_specs=[
            pl.BlockSpec((BLK, D_IN), lambda i: (i, 0)),
            pl.BlockSpec((BLK, D_META), lambda i: (i, 0)),
            pl.BlockSpec((D_IN + D_META, D_HYPER), lambda i: (0, 0)),
            pl.BlockSpec((D_IN, D_OUT * D_HYPER), lambda i: (0, 0)),
            pl.BlockSpec((D_HYPER, D_OUT), lambda i: (0, 0)),
            pl.BlockSpec((1, D_OUT), lambda i: (0, 0)),
            pl.BlockSpec((1, D_OUT), lambda i: (0, 0)),
        ],
        out_specs=[
            pl.BlockSpec((BLK, D_OUT), lambda i: (i, 0)),
            pl.BlockSpec((BLK, 8), lambda i: (i, 0)),
        ],
        out_shape=[
            jax.ShapeDtypeStruct((N_NODES, D_OUT), jnp.float32),
            jax.ShapeDtypeStruct((N_NODES, 8), jnp.float32),
        ],
    )(x, meta_k, wfT, g2, waT, aio, ajo)


def _sc_edge_body(src_h, dst_h, ai_h, aj_h, c_h, xt_h, zeros_h,
                  num_out, den_out,
                  ai_t, aj_t, c_t, den_t, sidx, didx, ebuf, rows, sem, num_sp):
    cid = lax.axis_index("c")
    sid = lax.axis_index("s")
    wid = sid * NC + cid
    pltpu.sync_copy(ai_h, ai_t)
    pltpu.sync_copy(aj_h, aj_t)
    pltpu.sync_copy(c_h, c_t)

    zv = jnp.zeros((VEC,), jnp.float32)

    def zbody(i, carry):
        den_t[pl.ds(i * VEC, VEC)] = zv
        return carry

    lax.fori_loop(0, N_NODES // VEC, zbody, 0)

    @pl.when(sid == 0)
    def _():
        pltpu.sync_copy(zeros_h, num_sp)

    plsc.subcore_barrier()

    def chunk_body(j, carry):
        roff = wid * ROWS_W + j * KSUB
        pltpu.sync_copy(src_h.at[pl.ds(roff, KSUB)], sidx)
        pltpu.sync_copy(dst_h.at[pl.ds(roff, KSUB)], didx)
        copies = [pltpu.async_copy(xt_h.at[sidx.at[b]], rows.at[b], sem)
                  for b in range(KSUB)]

        for b in range(KSUB):
            def ebody(q, carry2, b=b):
                s16 = sidx[b, pl.ds(q * VEC, VEC)]
                d16 = didx[b, pl.ds(q * VEC, VEC)]
                aiv = plsc.load_gather(ai_t, [d16])
                cv = plsc.load_gather(c_t, [d16])
                ajv = plsc.load_gather(aj_t, [s16])
                al = aiv + ajv
                al = jnp.where(al > 0, al, al * SLOPE)
                e = jnp.exp(jnp.minimum(al - cv, 60.0))
                e = jnp.where(s16 != d16, e, 0.0)
                plsc.addupdate_scatter(den_t, [d16], e)
                ebuf[pl.ds(b * SUB + q * VEC, VEC)] = e
                return carry2

            lax.fori_loop(0, SUB // VEC, ebody, 0)

        for cp in copies:
            cp.wait()

        for b in range(KSUB):
            def rbody(rr, carry2, b=b):
                ev = plsc.load_gather(
                    ebuf, [jnp.full((VEC,), b * SUB + rr, jnp.int32)])
                for cc in range(D_OUT // VEC):
                    sl = pl.ds(cc * VEC, VEC)
                    rows[b, rr, sl] = rows[b, rr, sl] * ev
                return carry2

            lax.fori_loop(0, SUB, rbody, 0)

        for b in range(KSUB):
            pltpu.sync_copy(rows.at[b], num_sp.at[didx.at[b]], add=True)
        return carry

    lax.fori_loop(0, NCHUNK, chunk_body, 0)

    pltpu.sync_copy(den_t, den_out.at[wid])
    plsc.subcore_barrier()

    @pl.when(sid == 0)
    def _():
        pltpu.sync_copy(num_sp, num_out.at[cid])


def _sc_edge_call(src2, dst2, ai, aj, c, xt, zeros):
    mesh = plsc.VectorSubcoreMesh(core_axis_name="c", subcore_axis_name="s")
    kfn = pl.kernel(
        _sc_edge_body,
        out_type=[
            jax.ShapeDtypeStruct((NC, N_NODES, D_OUT), jnp.float32),
            jax.ShapeDtypeStruct((NW, N_NODES), jnp.float32),
        ],
        mesh=mesh,
        scratch_types=[
            pltpu.VMEM((N_NODES,), jnp.float32),
            pltpu.VMEM((N_NODES,), jnp.float32),
            pltpu.VMEM((N_NODES,), jnp.float32),
            pltpu.VMEM((N_NODES,), jnp.float32),
            pltpu.VMEM((KSUB, SUB), jnp.int32),
            pltpu.VMEM((KSUB, SUB), jnp.int32),
            pltpu.VMEM((CHUNK,), jnp.float32),
            pltpu.VMEM((KSUB, SUB, D_OUT), jnp.float32),
            pltpu.SemaphoreType.DMA,
            pltpu.VMEM_SHARED((N_NODES, D_OUT), jnp.float32),
        ],
        compiler_params=pltpu.CompilerParams(needs_layout_passes=False,
                                             use_tc_tiling_on_sc=False),
    )
    return kfn(src2, dst2, ai, aj, c, xt, zeros)


def _finish_body(xt_ref, num_ref, den_ref, out_ref):
    num = num_ref[...]
    total = num[0] + num[1] + xt_ref[...]
    den = den_ref[...].sum(axis=1) + 1.0
    r = total / den[:, None]
    out_ref[...] = jnp.where(r > 0, r, jnp.exp(r) - 1.0)


def _run_finish(xt, num2, den32):
    grid = N_NODES // BLK
    return pl.pallas_call(
        _finish_body,
        grid=(grid,),
        in_specs=[
            pl.BlockSpec((BLK, D_OUT), lambda i: (i, 0)),
            pl.BlockSpec((2, BLK, D_OUT), lambda i: (0, i, 0)),
            pl.BlockSpec((BLK, NW), lambda i: (i, 0)),
        ],
        out_specs=pl.BlockSpec((BLK, D_OUT), lambda i: (i, 0)),
        out_shape=jax.ShapeDtypeStruct((N_NODES, D_OUT), jnp.float32),
    )(xt, num2, den32)


def kernel(meta_k, node_emb, W_f, W_hyper, W_att, att_i_o, att_j_o, edge_index):
    x = node_emb
    wfT = W_f.T
    g2 = (W_hyper.reshape(D_OUT, D_IN, D_HYPER)
          .transpose(1, 0, 2).reshape(D_IN, D_OUT * D_HYPER))
    waT = W_att[:D_OUT].T
    aio = att_i_o.reshape(1, D_OUT)
    ajo = att_j_o.reshape(1, D_OUT)
    xt, abc = _run_hyper(x, meta_k, wfT, g2, waT, aio, ajo)
    ai, aj, c = abc[:, 0], abc[:, 1], abc[:, 2]
    pad = jnp.zeros((2, E_PAD - N_EDGES), jnp.int32)
    ei = jnp.concatenate([edge_index, pad], axis=1)
    src2 = ei[0].reshape(ROWS_TOT, SUB)
    dst2 = ei[1].reshape(ROWS_TOT, SUB)
    zeros = jnp.zeros((N_NODES, D_OUT), jnp.float32)
    num2, den32 = _sc_edge_call(src2, dst2, ai, aj, c, xt, zeros)
    return _run_finish(xt, num2, den32.T)
